# baseline (speedup 1.0000x reference)
import functools

import jax
import jax.numpy as jnp
from jax import lax
from jax.experimental import pallas as pl
from jax.experimental.pallas import tpu as pltpu
from jax.experimental.pallas import tpu_sc as plsc

NC = 2
NS = 16
NW = NC * NS
DCH = 128
SCH = 32
RB = 4
NP = 2
DEGW = 16


def _mesh():
    return plsc.VectorSubcoreMesh(core_axis_name="c", subcore_axis_name="s")


def _sc_degree(dstp, n, npad, t):
    def body(dst_hbm, deg_out, deg_sh, ones_v, zero_v, dst_v):
        cid = lax.axis_index("c")
        sid = lax.axis_index("s")
        wid = sid * NC + cid

        def fill(i, _):
            ones_v[i, :] = jnp.full((DEGW,), 1.0, jnp.float32)
            zero_v[i, :] = jnp.zeros((DEGW,), jnp.float32)
            return _

        lax.fori_loop(0, DCH, fill, 0)

        def zinit(j, _):
            pltpu.sync_copy(zero_v, deg_sh.at[pl.ds(sid * (npad // NS) + j * DCH, DCH)])
            return _

        lax.fori_loop(0, npad // NS // DCH, zinit, 0)
        plsc.subcore_barrier()

        base = wid * t * DCH

        def step(i, _):
            pltpu.sync_copy(dst_hbm.at[pl.ds(base + i * DCH, DCH)], dst_v)
            pltpu.sync_copy(ones_v, deg_sh.at[dst_v], add=True)
            return _

        lax.fori_loop(0, t, step, 0)
        plsc.subcore_barrier()

        cp = npad // NS
        pltpu.sync_copy(deg_sh.at[pl.ds(sid * cp, cp)],
                        deg_out.at[cid, pl.ds(sid * cp, cp)])

    return pl.kernel(
        body,
        out_type=jax.ShapeDtypeStruct((NC, npad, DEGW), jnp.float32),
        mesh=_mesh(),
        scratch_types=[
            pltpu.VMEM_SHARED((npad, DEGW), jnp.float32),
            pltpu.VMEM((DCH, DEGW), jnp.float32),
            pltpu.VMEM((DCH, DEGW), jnp.float32),
            pltpu.VMEM((DCH,), jnp.int32),
        ],
    )(dstp)


def _sc_scatter(hp, srcp, dst2, n, npad, t, d):
    hc = t // NP
    assert t % NP == 0 and hc % RB == 0

    def body(hp_hbm, src_hbm, dst_hbm, acc_out, *refs):
        acc_sh, srcv, dstv, isem = refs[0], refs[1], refs[2], refs[3]
        rows = refs[4:4 + RB]
        gsem = refs[4 + RB:4 + 2 * RB]
        ssem = refs[4 + 2 * RB:4 + 3 * RB]
        cid = lax.axis_index("c")
        sid = lax.axis_index("s")
        wid = sid * NC + cid

        def zrow(i, _):
            rows[0][i // 8, pl.ds((i % 8) * 16, 16)] = jnp.zeros((16,), jnp.float32)
            return _

        lax.fori_loop(0, SCH * (d // 16), zrow, 0)

        nz = npad // NS // SCH
        for j in range(nz):
            pltpu.async_copy(rows[0], acc_sh.at[pl.ds(sid * (npad // NS) + j * SCH, SCH)], isem)
        for j in range(nz):
            pltpu.make_async_copy(rows[0], acc_sh.at[pl.ds(0, SCH)], isem).wait()
        plsc.subcore_barrier()

        for p in range(NP):
            pltpu.async_copy(
                src_hbm.at[pl.ds((wid * t + p * hc) * SCH, hc * SCH)], srcv, isem)
            pltpu.async_copy(dst_hbm.at[pl.ds(wid * t + p * hc, hc)], dstv, isem)
            pltpu.make_async_copy(
                src_hbm.at[pl.ds(0, hc * SCH)], srcv, isem).wait()
            pltpu.make_async_copy(dst_hbm.at[pl.ds(0, hc)], dstv, isem).wait()

            for b in range(RB - 1):
                pltpu.async_copy(hp_hbm.at[srcv.at[pl.ds(b * SCH, SCH)]], rows[b], gsem[b])

            def outer(o, _):
                for b in range(RB):
                    tt = o * RB + b
                    pltpu.make_async_copy(
                        hp_hbm.at[srcv.at[pl.ds(0, SCH)]], rows[b], gsem[b]).wait()
                    bu = (b - 1) % RB
                    uu = tt + RB - 1

                    @pl.when(uu < hc)
                    def _refill():
                        pltpu.async_copy(
                            hp_hbm.at[srcv.at[pl.ds(uu * SCH, SCH)]], rows[bu], gsem[bu])
                return _

            lax.fori_loop(0, hc // RB, outer, 0)

        plsc.subcore_barrier()
        cp = npad // NS
        pltpu.sync_copy(acc_sh.at[pl.ds(sid * cp, cp)],
                        acc_out.at[cid, pl.ds(sid * cp, cp)])

    scratch = [
        pltpu.VMEM_SHARED((npad, d), jnp.float32),
        pltpu.VMEM((hc * SCH,), jnp.int32),
        pltpu.VMEM((hc, SCH), jnp.int32),
        pltpu.SemaphoreType.DMA,
    ]
    scratch += [pltpu.VMEM((SCH, d), jnp.float32) for _ in range(RB)]
    scratch += [pltpu.SemaphoreType.DMA for _ in range(2 * RB)]

    return pl.kernel(
        body,
        out_type=jax.ShapeDtypeStruct((NC, npad, d), jnp.float32),
        mesh=_mesh(),
        scratch_types=scratch,
    )(hp, srcp, dst2)


def _tc_mm_scale(x, w, deg_parts, bn):
    n, d_in = x.shape
    d_out = w.shape[1]

    def body(x_ref, w_ref, dp_ref, o_ref):
        h = jnp.dot(x_ref[...], w_ref[...], preferred_element_type=jnp.float32)
        deg = dp_ref[0, :, 0:1] + dp_ref[1, :, 0:1] + 1.0
        o_ref[...] = h * lax.rsqrt(deg)

    return pl.pallas_call(
        body,
        grid=(n // bn,),
        in_specs=[
            pl.BlockSpec((bn, d_in), lambda i: (i, 0)),
            pl.BlockSpec((d_in, d_out), lambda i: (0, 0)),
            pl.BlockSpec((NC, bn, DEGW), lambda i: (0, i, 0)),
        ],
        out_specs=pl.BlockSpec((bn, d_out), lambda i: (i, 0)),
        out_shape=jax.ShapeDtypeStruct((n, d_out), jnp.float32),
    )(x, w, deg_parts)


def _tc_combine(acc_parts, hp, deg_parts, b2, bn):
    n, d = hp.shape

    def body(ap_ref, hp_ref, dp_ref, b_ref, o_ref):
        deg = dp_ref[0, :, 0:1] + dp_ref[1, :, 0:1] + 1.0
        s = ap_ref[0] + ap_ref[1] + hp_ref[...]
        o_ref[...] = s * lax.rsqrt(deg) + b_ref[...]

    return pl.pallas_call(
        body,
        grid=(n // bn,),
        in_specs=[
            pl.BlockSpec((NC, bn, d), lambda i: (0, i, 0)),
            pl.BlockSpec((bn, d), lambda i: (i, 0)),
            pl.BlockSpec((NC, bn, DEGW), lambda i: (0, i, 0)),
            pl.BlockSpec((1, d), lambda i: (0, 0)),
        ],
        out_specs=pl.BlockSpec((bn, d), lambda i: (i, 0)),
        out_shape=jax.ShapeDtypeStruct((n, d), jnp.float32),
    )(acc_parts, hp, deg_parts, b2)


@jax.jit
def kernel(x, edge_index, W, b):
    n, d_in = x.shape
    d_out = W.shape[1]
    e = edge_index.shape[1]

    import math
    q = NW * math.lcm(DCH, SCH * NP * RB)
    ep = q * (-(-e // q))
    td = ep // (NW * DCH)
    ts = ep // (NW * SCH)
    src = edge_index[0]
    dst = edge_index[1]
    srcp = jnp.concatenate([src, jnp.zeros((ep - e,), jnp.int32)])
    dstp = jnp.concatenate([dst, jnp.full((ep - e,), n, jnp.int32)])
    dst2 = dstp.reshape(NW * ts, SCH)

    npad = NS * DCH * (-(-(n + 1) // (NS * DCH)))

    deg_parts = _sc_degree(dstp, n, npad, td)
    hp = _tc_mm_scale(x, W, deg_parts, bn=1000)
    acc_parts = _sc_scatter(hp, srcp, dst2, n, npad, ts, d_out)
    return _tc_combine(acc_parts, hp, deg_parts, b.reshape(1, d_out), bn=1000)


# speedup vs baseline: 3.0285x; 3.0285x over previous
import functools

import jax
import jax.numpy as jnp
from jax import lax
from jax.experimental import pallas as pl
from jax.experimental.pallas import tpu as pltpu
from jax.experimental.pallas import tpu_sc as plsc

NC = 2
NS = 16
NW = NC * NS
DCH = 128
SCH = 64
RB = 4
NP = 4
DEGW = 16


def _mesh():
    return plsc.VectorSubcoreMesh(core_axis_name="c", subcore_axis_name="s")


RBD = 4


def _sc_degree(dst2, n, npad, t):
    assert t % RBD == 0 and t // RBD >= 2

    def body(dst_hbm, deg_out, deg_sh, ones_v, zero_v, idx_v, *sems):
        cid = lax.axis_index("c")
        sid = lax.axis_index("s")
        wid = sid * NC + cid

        def fill(i, _):
            ones_v[i, :] = jnp.full((DEGW,), 1.0, jnp.float32)
            zero_v[i, :] = jnp.zeros((DEGW,), jnp.float32)
            return _

        lax.fori_loop(0, DCH, fill, 0)

        def zinit(j, _):
            pltpu.sync_copy(zero_v, deg_sh.at[pl.ds(sid * (npad // NS) + j * DCH, DCH)])
            return _

        lax.fori_loop(0, npad // NS // DCH, zinit, 0)
        pltpu.sync_copy(dst_hbm.at[pl.ds(wid * t, t)], idx_v)
        plsc.subcore_barrier()

        for b in range(RBD):
            pltpu.async_copy(ones_v, deg_sh.at[idx_v.at[b]], sems[b], add=True)

        def step(o, _):
            for b in range(RBD):
                pltpu.make_async_copy(ones_v, deg_sh.at[idx_v.at[0]], sems[b]).wait()
                pltpu.async_copy(ones_v, deg_sh.at[idx_v.at[o * RBD + b]], sems[b], add=True)
            return _

        lax.fori_loop(1, t // RBD, step, 0)
        for b in range(RBD):
            pltpu.make_async_copy(ones_v, deg_sh.at[idx_v.at[0]], sems[b]).wait()
        plsc.subcore_barrier()

        cp = npad // NS
        pltpu.sync_copy(deg_sh.at[pl.ds(sid * cp, cp)],
                        deg_out.at[cid, pl.ds(sid * cp, cp)])

    return pl.kernel(
        body,
        out_type=jax.ShapeDtypeStruct((NC, npad, DEGW), jnp.float32),
        mesh=_mesh(),
        scratch_types=[
            pltpu.VMEM_SHARED((npad, DEGW), jnp.float32),
            pltpu.VMEM((DCH, DEGW), jnp.float32),
            pltpu.VMEM((DCH, DEGW), jnp.float32),
            pltpu.VMEM((t, DCH), jnp.int32),
        ] + [pltpu.SemaphoreType.DMA for _ in range(RBD)],
    )(dst2)


def _sc_scatter(hp, srcp, dst2, n, npad, t, d):
    hc = t // NP
    assert t % NP == 0 and hc % RB == 0

    def body(hp_hbm, src_hbm, dst_hbm, acc_out, *refs):
        acc_sh, srcv, dstv, isem = refs[0], refs[1], refs[2], refs[3]
        rows = refs[4:4 + RB]
        gsem = refs[4 + RB:4 + 2 * RB]
        ssem = refs[4 + 2 * RB:4 + 3 * RB]
        cid = lax.axis_index("c")
        sid = lax.axis_index("s")
        wid = sid * NC + cid

        def zrow(i, _):
            rows[0][i // 8, pl.ds((i % 8) * 16, 16)] = jnp.zeros((16,), jnp.float32)
            return _

        lax.fori_loop(0, SCH * (d // 16), zrow, 0)

        nz = npad // NS // SCH
        for j in range(nz):
            pltpu.async_copy(rows[0], acc_sh.at[pl.ds(sid * (npad // NS) + j * SCH, SCH)], isem)
        for j in range(nz):
            pltpu.make_async_copy(rows[0], acc_sh.at[pl.ds(0, SCH)], isem).wait()
        plsc.subcore_barrier()

        for p in range(NP):
            pltpu.async_copy(
                src_hbm.at[pl.ds((wid * t + p * hc) * SCH, hc * SCH)], srcv, isem)
            pltpu.async_copy(dst_hbm.at[pl.ds(wid * t + p * hc, hc)], dstv, isem)
            pltpu.make_async_copy(
                src_hbm.at[pl.ds(0, hc * SCH)], srcv, isem).wait()
            pltpu.make_async_copy(dst_hbm.at[pl.ds(0, hc)], dstv, isem).wait()

            for b in range(RB - 1):
                pltpu.async_copy(hp_hbm.at[srcv.at[pl.ds(b * SCH, SCH)]], rows[b], gsem[b])

            def outer(o, _):
                for b in range(RB):
                    tt = o * RB + b
                    pltpu.make_async_copy(
                        hp_hbm.at[srcv.at[pl.ds(0, SCH)]], rows[b], gsem[b]).wait()
                    pltpu.async_copy(rows[b], acc_sh.at[dstv.at[tt]], ssem[b], add=True)
                    bu = (b - 1) % RB
                    uu = tt + RB - 1

                    @pl.when(uu < hc)
                    def _refill():
                        @pl.when(tt >= 1)
                        def _wait_prev():
                            pltpu.make_async_copy(
                                rows[bu], acc_sh.at[dstv.at[0]], ssem[bu]).wait()

                        pltpu.async_copy(
                            hp_hbm.at[srcv.at[pl.ds(uu * SCH, SCH)]], rows[bu], gsem[bu])
                return _

            lax.fori_loop(0, hc // RB, outer, 0)
            for b in range(RB):
                pltpu.make_async_copy(rows[b], acc_sh.at[dstv.at[0]], ssem[b]).wait()

        plsc.subcore_barrier()
        cp = npad // NS
        pltpu.sync_copy(acc_sh.at[pl.ds(sid * cp, cp)],
                        acc_out.at[cid, pl.ds(sid * cp, cp)])

    scratch = [
        pltpu.VMEM_SHARED((npad, d), jnp.float32),
        pltpu.VMEM((hc * SCH,), jnp.int32),
        pltpu.VMEM((hc, SCH), jnp.int32),
        pltpu.SemaphoreType.DMA,
    ]
    scratch += [pltpu.VMEM((SCH, d), jnp.float32) for _ in range(RB)]
    scratch += [pltpu.SemaphoreType.DMA for _ in range(2 * RB)]

    return pl.kernel(
        body,
        out_type=jax.ShapeDtypeStruct((NC, npad, d), jnp.float32),
        mesh=_mesh(),
        scratch_types=scratch,
    )(hp, srcp, dst2)


def _tc_mm_scale(x, w, deg_parts, bn):
    n, d_in = x.shape
    d_out = w.shape[1]

    def body(x_ref, w_ref, dp_ref, o_ref):
        h = jnp.dot(x_ref[...], w_ref[...], preferred_element_type=jnp.float32)
        deg = dp_ref[0, :, 0:1] + dp_ref[1, :, 0:1] + 1.0
        o_ref[...] = h * lax.rsqrt(deg)

    return pl.pallas_call(
        body,
        grid=(n // bn,),
        in_specs=[
            pl.BlockSpec((bn, d_in), lambda i: (i, 0)),
            pl.BlockSpec((d_in, d_out), lambda i: (0, 0)),
            pl.BlockSpec((NC, bn, DEGW), lambda i: (0, i, 0)),
        ],
        out_specs=pl.BlockSpec((bn, d_out), lambda i: (i, 0)),
        out_shape=jax.ShapeDtypeStruct((n, d_out), jnp.float32),
    )(x, w, deg_parts)


def _tc_combine(acc_parts, hp, deg_parts, b2, bn):
    n, d = hp.shape

    def body(ap_ref, hp_ref, dp_ref, b_ref, o_ref):
        deg = dp_ref[0, :, 0:1] + dp_ref[1, :, 0:1] + 1.0
        s = ap_ref[0] + ap_ref[1] + hp_ref[...]
        o_ref[...] = s * lax.rsqrt(deg) + b_ref[...]

    return pl.pallas_call(
        body,
        grid=(n // bn,),
        in_specs=[
            pl.BlockSpec((NC, bn, d), lambda i: (0, i, 0)),
            pl.BlockSpec((bn, d), lambda i: (i, 0)),
            pl.BlockSpec((NC, bn, DEGW), lambda i: (0, i, 0)),
            pl.BlockSpec((1, d), lambda i: (0, 0)),
        ],
        out_specs=pl.BlockSpec((bn, d), lambda i: (i, 0)),
        out_shape=jax.ShapeDtypeStruct((n, d), jnp.float32),
    )(acc_parts, hp, deg_parts, b2)


@jax.jit
def kernel(x, edge_index, W, b):
    n, d_in = x.shape
    d_out = W.shape[1]
    e = edge_index.shape[1]

    import math
    q = NW * math.lcm(DCH, SCH * NP * RB)
    ep = q * (-(-e // q))
    td = ep // (NW * DCH)
    ts = ep // (NW * SCH)
    src = edge_index[0]
    dst = edge_index[1]
    npad = NS * DCH * (-(-(n + 1) // (NS * DCH)))

    pad = jnp.arange(ep - e, dtype=jnp.int32)
    srcp = jnp.concatenate([src, pad % n])
    dstp = jnp.concatenate([dst, n + pad % (npad - n)])
    dst2 = dstp.reshape(NW * ts, SCH)

    dst2d = dstp.reshape(NW * td, DCH)
    deg_parts = _sc_degree(dst2d, n, npad, td)
    hp = _tc_mm_scale(x, W, deg_parts, bn=1000)
    acc_parts = _sc_scatter(hp, srcp, dst2, n, npad, ts, d_out)
    return _tc_combine(acc_parts, hp, deg_parts, b.reshape(1, d_out), bn=1000)
